# Initial kernel scaffold; baseline (speedup 1.0000x reference)
import jax
import jax.numpy as jnp
from jax.experimental import pallas as pl


def kernel(node_features, edge_node_indices, edge_features, W_edge, b_edge, W_node, b_node):
    raise NotImplementedError("write your pallas kernel here")


# speedup vs baseline: 6.4226x; 6.4226x over previous
import jax
import jax.numpy as jnp
from jax import lax
from jax.experimental import pallas as pl
from jax.experimental.pallas import tpu as pltpu
from jax.experimental.pallas import tpu_sc as plsc

N = 10000
M = 320000
DF = 128
DE = 16
DM = 32

NC = 2
NS = 16
NW = NC * NS
EB = 80
KB = M // (NW * EB)
ZR = 624
ZREM = N - NS * ZR


def _sc_body(a_hbm, b_hbm, e_hbm, src_hbm, dst_hbm, out_hbm,
             src_buf, dst_buf, a_buf, b_buf, e_buf, m_buf, zbuf, s_shared,
             sem_in0, sem_in1, sem_s0, sem_s1):
    c = lax.axis_index("c")
    s = lax.axis_index("s")
    w = c * NS + s
    row0 = w * KB

    sems_in = (sem_in0, sem_in1)
    sems_s = (sem_s0, sem_s1)

    pltpu.sync_copy(src_hbm.at[w], src_buf)
    pltpu.sync_copy(dst_hbm.at[w], dst_buf)

    zeros16 = jnp.zeros((16,), jnp.float32)

    def _zero_fill(i, _):
        zbuf[i, pl.ds(0, 16)] = zeros16
        zbuf[i, pl.ds(16, 16)] = zeros16
        return 0

    lax.fori_loop(0, ZR // 3, _zero_fill, 0)

    def _zero_copy(k, _):
        pltpu.sync_copy(zbuf, s_shared.at[pl.ds(s * ZR + k * (ZR // 3), ZR // 3)])
        return 0

    lax.fori_loop(0, 3, _zero_copy, 0)

    @pl.when(s == 0)
    def _zero_rem():
        pltpu.sync_copy(zbuf.at[pl.ds(0, ZREM)],
                        s_shared.at[pl.ds(NS * ZR, ZREM)])

    plsc.subcore_barrier()

    def _prefetch(j, b):
        erow = (row0 + j) * EB
        pltpu.async_copy(e_hbm.at[pl.ds(erow, EB)], e_buf.at[b], sems_in[b])
        pltpu.async_copy(a_hbm.at[src_buf.at[j]], a_buf.at[b], sems_in[b])
        pltpu.async_copy(b_hbm.at[dst_buf.at[j]], b_buf.at[b], sems_in[b])

    def _drain_in(b):
        pltpu.make_async_copy(e_hbm.at[pl.ds(0, EB)], e_buf.at[b], sems_in[b]).wait()
        pltpu.make_async_copy(e_hbm.at[pl.ds(0, EB)], a_buf.at[b], sems_in[b]).wait()
        pltpu.make_async_copy(e_hbm.at[pl.ds(0, EB)], b_buf.at[b], sems_in[b]).wait()

    def _drain_s(b):
        pltpu.make_async_copy(e_hbm.at[pl.ds(0, EB)], m_buf.at[b], sems_s[b]).wait()

    def _compute(b):
        @plsc.parallel_loop(0, EB, 1, unroll=4)
        def _rows(r):
            for h in range(DM // 16):
                sl = pl.ds(h * 16, 16)
                v = a_buf[b, r, sl] + b_buf[b, r, sl] + e_buf[b, r, sl]
                m_buf[b, r, sl] = jnp.maximum(v, 0.0)

    _prefetch(0, 0)
    _prefetch(1, 1)

    @pl.loop(0, KB - 1, step=2)
    def _round(j0):
        for b in range(2):
            j = j0 + b
            _drain_in(b)

            @pl.when(j >= 2)
            def _w1():
                _drain_s(b)

            _compute(b)
            pltpu.async_copy(m_buf.at[b], s_shared.at[src_buf.at[j]],
                             sems_s[b], add=True)

            @pl.when(j + 2 < KB)
            def _w2():
                _prefetch(j + 2, b)

    jl = KB - 1
    _drain_in(0)
    _drain_s(0)
    _compute(0)
    pltpu.async_copy(m_buf.at[0], s_shared.at[src_buf.at[jl]], sems_s[0],
                     add=True)
    _drain_s(0)
    _drain_s(1)

    plsc.subcore_barrier()
    pltpu.sync_copy(s_shared.at[pl.ds(s * ZR, ZR)],
                    out_hbm.at[c, pl.ds(s * ZR, ZR)])

    @pl.when(s == 0)
    def _write_rem():
        pltpu.sync_copy(s_shared.at[pl.ds(NS * ZR, ZREM)],
                        out_hbm.at[c, pl.ds(NS * ZR, ZREM)])


_SC_CACHE = {}


def _sc_scatter_fn():
    if "k" not in _SC_CACHE:
        _SC_CACHE["k"] = pl.kernel(
            _sc_body,
            out_type=jax.ShapeDtypeStruct((NC, N, DM), jnp.float32),
            mesh=plsc.VectorSubcoreMesh(core_axis_name="c",
                                        subcore_axis_name="s",
                                        num_cores=NC, num_subcores=NS),
            scratch_types=[
                pltpu.VMEM((KB, EB), jnp.int32),
                pltpu.VMEM((KB, EB), jnp.int32),
                pltpu.VMEM((2, EB, DM), jnp.float32),
                pltpu.VMEM((2, EB, DM), jnp.float32),
                pltpu.VMEM((2, EB, DM), jnp.float32),
                pltpu.VMEM((2, EB, DM), jnp.float32),
                pltpu.VMEM((ZR // 3, DM), jnp.float32),
                pltpu.VMEM_SHARED((N, DM), jnp.float32),
                pltpu.SemaphoreType.DMA,
                pltpu.SemaphoreType.DMA,
                pltpu.SemaphoreType.DMA,
                pltpu.SemaphoreType.DMA,
            ],
            compiler_params=pltpu.CompilerParams(use_tc_tiling_on_sc=False),
        )
    return _SC_CACHE["k"]


def _ab_body(x_ref, w_ref, oa_ref, ob_ref):
    p = jnp.dot(x_ref[...], w_ref[...], preferred_element_type=jnp.float32)
    oa_ref[...] = p[:, :DM]
    ob_ref[...] = p[:, DM:]


def _ab_call(x, w_cat):
    rb = 2000
    return pl.pallas_call(
        _ab_body,
        grid=(N // rb,),
        in_specs=[pl.BlockSpec((rb, DF), lambda i: (i, 0)),
                  pl.BlockSpec((DF, 2 * DM), lambda i: (0, 0))],
        out_specs=[pl.BlockSpec((rb, DM), lambda i: (i, 0)),
                   pl.BlockSpec((rb, DM), lambda i: (i, 0))],
        out_shape=[jax.ShapeDtypeStruct((N, DM), jnp.float32)] * 2,
    )(x, w_cat)


def _e_body(ef_ref, w_ref, b_ref, o_ref):
    o_ref[...] = (jnp.dot(ef_ref[...], w_ref[...],
                          preferred_element_type=jnp.float32) + b_ref[...])


def _e_call(ef, w_e, b_e):
    rb = 8000
    return pl.pallas_call(
        _e_body,
        grid=(M // rb,),
        in_specs=[pl.BlockSpec((rb, DE), lambda i: (i, 0)),
                  pl.BlockSpec((DE, DM), lambda i: (0, 0)),
                  pl.BlockSpec((1, DM), lambda i: (0, 0))],
        out_specs=pl.BlockSpec((rb, DM), lambda i: (i, 0)),
        out_shape=jax.ShapeDtypeStruct((M, DM), jnp.float32),
    )(ef, w_e, b_e)


def _post_body(x_ref, s_ref, wx_ref, ws_ref, b_ref, o_ref):
    ssum = s_ref[0] + s_ref[1]
    acc = jnp.dot(x_ref[...], wx_ref[...], preferred_element_type=jnp.float32)
    acc = acc + jnp.dot(ssum, ws_ref[...], preferred_element_type=jnp.float32)
    o_ref[...] = jnp.maximum(acc + b_ref[...], 0.0)


def _post_call(x, s2, w_x, w_s, b_n):
    rb = 2000
    return pl.pallas_call(
        _post_body,
        grid=(N // rb,),
        in_specs=[pl.BlockSpec((rb, DF), lambda i: (i, 0)),
                  pl.BlockSpec((NC, rb, DM), lambda i: (0, i, 0)),
                  pl.BlockSpec((DF, DF), lambda i: (0, 0)),
                  pl.BlockSpec((DM, DF), lambda i: (0, 0)),
                  pl.BlockSpec((1, DF), lambda i: (0, 0))],
        out_specs=pl.BlockSpec((rb, DF), lambda i: (i, 0)),
        out_shape=jax.ShapeDtypeStruct((N, DF), jnp.float32),
    )(x, s2, w_x, w_s, b_n)


def kernel(node_features, edge_node_indices, edge_features,
           W_edge, b_edge, W_node, b_node):
    src = edge_node_indices[0].astype(jnp.int32).reshape(NW, KB, EB)
    dst = edge_node_indices[1].astype(jnp.int32).reshape(NW, KB, EB)
    w_cat = jnp.concatenate([W_edge[:DF], W_edge[DF:2 * DF]], axis=1)
    a, b = _ab_call(node_features, w_cat)
    e = _e_call(edge_features, W_edge[2 * DF:], b_edge.reshape(1, DM))
    s2 = _sc_scatter_fn()(a, b, e, src, dst)
    return _post_call(node_features, s2, W_node[:DF], W_node[DF:],
                      b_node.reshape(1, DF))


# speedup vs baseline: 9.2616x; 1.4420x over previous
import jax
import jax.numpy as jnp
from jax import lax
from jax.experimental import pallas as pl
from jax.experimental.pallas import tpu as pltpu
from jax.experimental.pallas import tpu_sc as plsc

N = 10000
M = 320000
DF = 128
DE = 16
DM = 32

NC = 2
NS = 16
NW = NC * NS
EB = 80
KB = M // (NW * EB)
ZR = 624
ZREM = N - NS * ZR


def _sc_body(a_hbm, b_hbm, e_hbm, idx_hbm, out_hbm,
             src_buf, dst_buf, a_buf, b_buf, e_buf, m_buf, zbuf, s_shared,
             sem_in0, sem_in1, sem_s0, sem_s1):
    c = lax.axis_index("c")
    s = lax.axis_index("s")
    w = c * NS + s
    row0 = w * KB

    sems_in = (sem_in0, sem_in1)
    sems_s = (sem_s0, sem_s1)

    pltpu.sync_copy(idx_hbm.at[0, pl.ds(w * KB * EB, KB * EB)], src_buf)
    pltpu.sync_copy(idx_hbm.at[1, pl.ds(w * KB * EB, KB * EB)], dst_buf)

    zeros16 = jnp.zeros((16,), jnp.float32)

    def _zero_fill(i, _):
        zbuf[i, pl.ds(0, 16)] = zeros16
        zbuf[i, pl.ds(16, 16)] = zeros16
        return 0

    lax.fori_loop(0, ZR // 3, _zero_fill, 0)

    def _zero_copy(k, _):
        pltpu.sync_copy(zbuf, s_shared.at[pl.ds(s * ZR + k * (ZR // 3), ZR // 3)])
        return 0

    lax.fori_loop(0, 3, _zero_copy, 0)

    @pl.when(s == 0)
    def _zero_rem():
        pltpu.sync_copy(zbuf.at[pl.ds(0, ZREM)],
                        s_shared.at[pl.ds(NS * ZR, ZREM)])

    plsc.subcore_barrier()

    def _prefetch(j, b):
        erow = (row0 + j) * EB
        pltpu.async_copy(e_hbm.at[pl.ds(erow, EB)], e_buf.at[b], sems_in[b])
        pltpu.async_copy(a_hbm.at[src_buf.at[pl.ds(j * EB, EB)]], a_buf.at[b],
                         sems_in[b])
        pltpu.async_copy(b_hbm.at[dst_buf.at[pl.ds(j * EB, EB)]], b_buf.at[b],
                         sems_in[b])

    def _drain_in(b):
        pltpu.make_async_copy(e_hbm.at[pl.ds(0, EB)], e_buf.at[b], sems_in[b]).wait()
        pltpu.make_async_copy(e_hbm.at[pl.ds(0, EB)], a_buf.at[b], sems_in[b]).wait()
        pltpu.make_async_copy(e_hbm.at[pl.ds(0, EB)], b_buf.at[b], sems_in[b]).wait()

    def _drain_s(b):
        pltpu.make_async_copy(e_hbm.at[pl.ds(0, EB)], m_buf.at[b], sems_s[b]).wait()

    def _compute(b):
        @plsc.parallel_loop(0, EB, 1, unroll=4)
        def _rows(r):
            for h in range(DM // 16):
                sl = pl.ds(h * 16, 16)
                v = a_buf[b, r, sl] + b_buf[b, r, sl] + e_buf[b, r, sl]
                m_buf[b, r, sl] = jnp.maximum(v, 0.0)

    _prefetch(0, 0)
    _prefetch(1, 1)

    @pl.loop(0, KB - 1, step=2)
    def _round(j0):
        for b in range(2):
            j = j0 + b
            _drain_in(b)

            @pl.when(j >= 2)
            def _w1():
                _drain_s(b)

            _compute(b)
            pltpu.async_copy(m_buf.at[b],
                             s_shared.at[src_buf.at[pl.ds(j * EB, EB)]],
                             sems_s[b], add=True)

            @pl.when(j + 2 < KB)
            def _w2():
                _prefetch(j + 2, b)

    jl = KB - 1
    _drain_in(0)
    _drain_s(0)
    _compute(0)
    pltpu.async_copy(m_buf.at[0],
                     s_shared.at[src_buf.at[pl.ds(jl * EB, EB)]], sems_s[0],
                     add=True)
    _drain_s(0)
    _drain_s(1)

    plsc.subcore_barrier()
    pltpu.sync_copy(s_shared.at[pl.ds(s * ZR, ZR)],
                    out_hbm.at[c, pl.ds(s * ZR, ZR)])

    @pl.when(s == 0)
    def _write_rem():
        pltpu.sync_copy(s_shared.at[pl.ds(NS * ZR, ZREM)],
                        out_hbm.at[c, pl.ds(NS * ZR, ZREM)])


_SC_CACHE = {}


def _sc_scatter_fn():
    if "k" not in _SC_CACHE:
        _SC_CACHE["k"] = pl.kernel(
            _sc_body,
            out_type=jax.ShapeDtypeStruct((NC, N, DM), jnp.float32),
            mesh=plsc.VectorSubcoreMesh(core_axis_name="c",
                                        subcore_axis_name="s",
                                        num_cores=NC, num_subcores=NS),
            scratch_types=[
                pltpu.VMEM((KB * EB,), jnp.int32),
                pltpu.VMEM((KB * EB,), jnp.int32),
                pltpu.VMEM((2, EB, DM), jnp.float32),
                pltpu.VMEM((2, EB, DM), jnp.float32),
                pltpu.VMEM((2, EB, DM), jnp.float32),
                pltpu.VMEM((2, EB, DM), jnp.float32),
                pltpu.VMEM((ZR // 3, DM), jnp.float32),
                pltpu.VMEM_SHARED((N, DM), jnp.float32),
                pltpu.SemaphoreType.DMA,
                pltpu.SemaphoreType.DMA,
                pltpu.SemaphoreType.DMA,
                pltpu.SemaphoreType.DMA,
            ],
            compiler_params=pltpu.CompilerParams(use_tc_tiling_on_sc=False),
        )
    return _SC_CACHE["k"]


def _ab_body(x_ref, w_ref, oa_ref, ob_ref):
    p = jnp.dot(x_ref[...], w_ref[...], preferred_element_type=jnp.float32)
    oa_ref[...] = p[:, :DM]
    ob_ref[...] = p[:, DM:]


def _ab_call(x, w_cat):
    rb = 2000
    return pl.pallas_call(
        _ab_body,
        grid=(N // rb,),
        in_specs=[pl.BlockSpec((rb, DF), lambda i: (i, 0)),
                  pl.BlockSpec((DF, 2 * DM), lambda i: (0, 0))],
        out_specs=[pl.BlockSpec((rb, DM), lambda i: (i, 0)),
                   pl.BlockSpec((rb, DM), lambda i: (i, 0))],
        out_shape=[jax.ShapeDtypeStruct((N, DM), jnp.float32)] * 2,
    )(x, w_cat)


def _e_body(ef_ref, w_ref, b_ref, o_ref):
    o_ref[...] = (jnp.dot(ef_ref[...], w_ref[...],
                          preferred_element_type=jnp.float32) + b_ref[...])


def _e_call(ef8, w_blk, b_blk):
    rows = M // 8
    rb = 4000
    return pl.pallas_call(
        _e_body,
        grid=(rows // rb,),
        in_specs=[pl.BlockSpec((rb, 8 * DE), lambda i: (i, 0)),
                  pl.BlockSpec((8 * DE, 8 * DM), lambda i: (0, 0)),
                  pl.BlockSpec((1, 8 * DM), lambda i: (0, 0))],
        out_specs=pl.BlockSpec((rb, 8 * DM), lambda i: (i, 0)),
        out_shape=jax.ShapeDtypeStruct((rows, 8 * DM), jnp.float32),
    )(ef8, w_blk, b_blk)


def _post_body(x_ref, s_ref, wx_ref, ws_ref, b_ref, o_ref):
    ssum = s_ref[0] + s_ref[1]
    acc = jnp.dot(x_ref[...], wx_ref[...], preferred_element_type=jnp.float32)
    acc = acc + jnp.dot(ssum, ws_ref[...], preferred_element_type=jnp.float32)
    o_ref[...] = jnp.maximum(acc + b_ref[...], 0.0)


def _post_call(x, s2, w_x, w_s, b_n):
    rb = 2000
    return pl.pallas_call(
        _post_body,
        grid=(N // rb,),
        in_specs=[pl.BlockSpec((rb, DF), lambda i: (i, 0)),
                  pl.BlockSpec((NC, rb, DM), lambda i: (0, i, 0)),
                  pl.BlockSpec((DF, DF), lambda i: (0, 0)),
                  pl.BlockSpec((DM, DF), lambda i: (0, 0)),
                  pl.BlockSpec((1, DF), lambda i: (0, 0))],
        out_specs=pl.BlockSpec((rb, DF), lambda i: (i, 0)),
        out_shape=jax.ShapeDtypeStruct((N, DF), jnp.float32),
    )(x, s2, w_x, w_s, b_n)


def kernel(node_features, edge_node_indices, edge_features,
           W_edge, b_edge, W_node, b_node):
    idx = edge_node_indices.astype(jnp.int32)
    w_cat = jnp.concatenate([W_edge[:DF], W_edge[DF:2 * DF]], axis=1)
    w_blk = jnp.kron(jnp.eye(8, dtype=jnp.float32), W_edge[2 * DF:])
    b_blk = jnp.tile(b_edge, 8).reshape(1, 8 * DM)
    a, b = _ab_call(node_features, w_cat)
    e8 = _e_call(edge_features.reshape(M // 8, 8 * DE), w_blk, b_blk)
    s2 = _sc_scatter_fn()(a, b, e8.reshape(M, DM), idx)
    return _post_call(node_features, s2, W_node[:DF], W_node[DF:],
                      b_node.reshape(1, DF))


# speedup vs baseline: 10.9100x; 1.1780x over previous
import jax
import jax.numpy as jnp
from jax import lax
from jax.experimental import pallas as pl
from jax.experimental.pallas import tpu as pltpu
from jax.experimental.pallas import tpu_sc as plsc

N = 10000
M = 320000
DF = 128
DE = 16
DM = 32

NC = 2
NS = 16
NW = NC * NS
EB = 80
KB = M // (NW * EB)
ZR = 624
ZREM = N - NS * ZR


def _sc_body(a_hbm, b_hbm, e_hbm, idx_hbm, out_hbm,
             src_buf, dst_buf, a_buf, b_buf, e_buf, m_buf, zbuf, s_shared,
             sem_in0, sem_in1, sem_s0, sem_s1):
    c = lax.axis_index("c")
    s = lax.axis_index("s")
    w = c * NS + s
    row0 = w * KB

    sems_in = (sem_in0, sem_in1)
    sems_s = (sem_s0, sem_s1)

    pltpu.sync_copy(idx_hbm.at[0, pl.ds(w * KB * EB, KB * EB)], src_buf)
    pltpu.sync_copy(idx_hbm.at[1, pl.ds(w * KB * EB, KB * EB)], dst_buf)

    zeros16 = jnp.zeros((16,), jnp.float32)

    def _zero_fill(i, _):
        zbuf[i, pl.ds(0, 16)] = zeros16
        zbuf[i, pl.ds(16, 16)] = zeros16
        return 0

    lax.fori_loop(0, ZR // 3, _zero_fill, 0)

    def _zero_copy(k, _):
        pltpu.sync_copy(zbuf, s_shared.at[pl.ds(s * ZR + k * (ZR // 3), ZR // 3)])
        return 0

    lax.fori_loop(0, 3, _zero_copy, 0)

    @pl.when(s == 0)
    def _zero_rem():
        pltpu.sync_copy(zbuf.at[pl.ds(0, ZREM)],
                        s_shared.at[pl.ds(NS * ZR, ZREM)])

    plsc.subcore_barrier()

    def _prefetch(j, b):
        erow = (row0 + j) * EB
        pltpu.async_copy(e_hbm.at[pl.ds(erow, EB)], e_buf.at[b], sems_in[b])
        pltpu.async_copy(a_hbm.at[src_buf.at[pl.ds(j * EB, EB)]], a_buf.at[b],
                         sems_in[b])
        pltpu.async_copy(b_hbm.at[dst_buf.at[pl.ds(j * EB, EB)]], b_buf.at[b],
                         sems_in[b])

    def _drain_in(b):
        pltpu.make_async_copy(e_hbm.at[pl.ds(0, EB)], e_buf.at[b], sems_in[b]).wait()
        pltpu.make_async_copy(e_hbm.at[pl.ds(0, EB)], a_buf.at[b], sems_in[b]).wait()
        pltpu.make_async_copy(e_hbm.at[pl.ds(0, EB)], b_buf.at[b], sems_in[b]).wait()

    def _drain_s(b):
        pltpu.make_async_copy(e_hbm.at[pl.ds(0, EB)], m_buf.at[b], sems_s[b]).wait()

    def _compute(b):
        @plsc.parallel_loop(0, EB, 1, unroll=4)
        def _rows(r):
            for h in range(DM // 16):
                sl = pl.ds(h * 16, 16)
                v = a_buf[b, r, sl] + b_buf[b, r, sl] + e_buf[b, r, sl]
                m_buf[b, r, sl] = jnp.maximum(v, 0.0)

    _prefetch(0, 0)
    _prefetch(1, 1)

    @pl.loop(0, KB - 1, step=2)
    def _round(j0):
        for b in range(2):
            j = j0 + b
            _drain_in(b)

            @pl.when(j >= 2)
            def _w1():
                _drain_s(b)

            _compute(b)
            pltpu.async_copy(m_buf.at[b],
                             s_shared.at[src_buf.at[pl.ds(j * EB, EB)]],
                             sems_s[b], add=True)

            @pl.when(j + 2 < KB)
            def _w2():
                _prefetch(j + 2, b)

    jl = KB - 1
    _drain_in(0)
    _drain_s(0)
    _compute(0)
    pltpu.async_copy(m_buf.at[0],
                     s_shared.at[src_buf.at[pl.ds(jl * EB, EB)]], sems_s[0],
                     add=True)
    _drain_s(0)
    _drain_s(1)

    plsc.subcore_barrier()
    pltpu.sync_copy(s_shared.at[pl.ds(s * ZR, ZR)],
                    out_hbm.at[c, pl.ds(s * ZR, ZR)])

    @pl.when(s == 0)
    def _write_rem():
        pltpu.sync_copy(s_shared.at[pl.ds(NS * ZR, ZREM)],
                        out_hbm.at[c, pl.ds(NS * ZR, ZREM)])


_SC_CACHE = {}


def _sc_scatter_fn():
    if "k" not in _SC_CACHE:
        _SC_CACHE["k"] = pl.kernel(
            _sc_body,
            out_type=jax.ShapeDtypeStruct((NC, N, DM), jnp.float32),
            mesh=plsc.VectorSubcoreMesh(core_axis_name="c",
                                        subcore_axis_name="s",
                                        num_cores=NC, num_subcores=NS),
            scratch_types=[
                pltpu.VMEM((KB * EB,), jnp.int32),
                pltpu.VMEM((KB * EB,), jnp.int32),
                pltpu.VMEM((2, EB, DM), jnp.float32),
                pltpu.VMEM((2, EB, DM), jnp.float32),
                pltpu.VMEM((2, EB, DM), jnp.float32),
                pltpu.VMEM((2, EB, DM), jnp.float32),
                pltpu.VMEM((ZR // 3, DM), jnp.float32),
                pltpu.VMEM_SHARED((N, DM), jnp.float32),
                pltpu.SemaphoreType.DMA,
                pltpu.SemaphoreType.DMA,
                pltpu.SemaphoreType.DMA,
                pltpu.SemaphoreType.DMA,
            ],
            compiler_params=pltpu.CompilerParams(use_tc_tiling_on_sc=False),
        )
    return _SC_CACHE["k"]


def _ab_body(x_ref, w_ref, oa_ref, ob_ref):
    p = jnp.dot(x_ref[...], w_ref[...], preferred_element_type=jnp.float32)
    oa_ref[...] = p[:, :DM]
    ob_ref[...] = p[:, DM:]


def _ab_call(x, w_cat):
    rb = 2000
    return pl.pallas_call(
        _ab_body,
        grid=(N // rb,),
        in_specs=[pl.BlockSpec((rb, DF), lambda i: (i, 0)),
                  pl.BlockSpec((DF, 2 * DM), lambda i: (0, 0))],
        out_specs=[pl.BlockSpec((rb, DM), lambda i: (i, 0)),
                   pl.BlockSpec((rb, DM), lambda i: (i, 0))],
        out_shape=[jax.ShapeDtypeStruct((N, DM), jnp.float32)] * 2,
    )(x, w_cat)


_EC = 6400


def _e_body(eft_ref, w_ref, b_ref, o_ref):
    p = lax.dot_general(eft_ref[...], w_ref[...],
                        (((0,), (0,)), ((), ())),
                        preferred_element_type=jnp.float32)
    p = p + b_ref[...]
    p3 = p.reshape(_EC // 8, 8, DM)
    for k in range(8):
        o_ref[:, k * DM:(k + 1) * DM] = p3[:, k, :]


def _e_call(eft, w_e, b_e):
    rows = M // 8
    return pl.pallas_call(
        _e_body,
        grid=(M // _EC,),
        in_specs=[pl.BlockSpec((DE, _EC), lambda i: (0, i)),
                  pl.BlockSpec((DE, DM), lambda i: (0, 0)),
                  pl.BlockSpec((1, DM), lambda i: (0, 0))],
        out_specs=pl.BlockSpec((_EC // 8, 8 * DM), lambda i: (i, 0)),
        out_shape=jax.ShapeDtypeStruct((rows, 8 * DM), jnp.float32),
    )(eft, w_e, b_e)


def _post_body(x_ref, s_ref, wx_ref, ws_ref, b_ref, o_ref):
    ssum = s_ref[0] + s_ref[1]
    acc = jnp.dot(x_ref[...], wx_ref[...], preferred_element_type=jnp.float32)
    acc = acc + jnp.dot(ssum, ws_ref[...], preferred_element_type=jnp.float32)
    o_ref[...] = jnp.maximum(acc + b_ref[...], 0.0)


def _post_call(x, s2, w_x, w_s, b_n):
    rb = 2000
    return pl.pallas_call(
        _post_body,
        grid=(N // rb,),
        in_specs=[pl.BlockSpec((rb, DF), lambda i: (i, 0)),
                  pl.BlockSpec((NC, rb, DM), lambda i: (0, i, 0)),
                  pl.BlockSpec((DF, DF), lambda i: (0, 0)),
                  pl.BlockSpec((DM, DF), lambda i: (0, 0)),
                  pl.BlockSpec((1, DF), lambda i: (0, 0))],
        out_specs=pl.BlockSpec((rb, DF), lambda i: (i, 0)),
        out_shape=jax.ShapeDtypeStruct((N, DF), jnp.float32),
    )(x, s2, w_x, w_s, b_n)


def kernel(node_features, edge_node_indices, edge_features,
           W_edge, b_edge, W_node, b_node):
    idx = edge_node_indices.astype(jnp.int32)
    w_cat = jnp.concatenate([W_edge[:DF], W_edge[DF:2 * DF]], axis=1)
    a, b = _ab_call(node_features, w_cat)
    e8 = _e_call(edge_features.T, W_edge[2 * DF:], b_edge.reshape(1, DM))
    s2 = _sc_scatter_fn()(a, b, e8.reshape(M, DM), idx)
    return _post_call(node_features, s2, W_node[:DF], W_node[DF:],
                      b_node.reshape(1, DF))
